# baseline (speedup 1.0000x reference)
import functools

import jax
import jax.numpy as jnp
from jax import lax
from jax.experimental import pallas as pl
from jax.experimental.pallas import tpu as pltpu
from jax.experimental.pallas import tpu_sc as plsc

N = 4096
K = 8192
D = 32

BN = 4096
BK = 2048


AR = 8


def _argmin_body(xt_ref, e_ref, idx_ref, esq_ref, kio_ref):
    @pl.when(pl.program_id(0) == 0)
    def _():
        e_all = e_ref[...]
        esq_ref[...] = jnp.sum(e_all * e_all, axis=1, keepdims=True)
        kio_ref[...] = lax.broadcasted_iota(
            jnp.int32, (K, 1), 0).astype(jnp.float32)

    xt = xt_ref[...]
    x2t = xt * -2.0
    x_sq = jnp.broadcast_to(
        jnp.sum(xt * xt, axis=0, keepdims=True), (AR, BN))
    run_min = jnp.full((AR, BN), jnp.inf, dtype=jnp.float32)
    run_idx = jnp.full((AR, BN), float(K), dtype=jnp.float32)
    for kc in range(K // BK):
        e_c = e_ref[kc * BK:(kc + 1) * BK, :]
        m2 = jnp.dot(e_c, x2t, preferred_element_type=jnp.float32)
        m3 = m2.reshape(BK // AR, AR, BN)
        e3 = esq_ref[kc * BK:(kc + 1) * BK, :].reshape(BK // AR, AR, 1)
        k3 = kio_ref[kc * BK:(kc + 1) * BK, :].reshape(BK // AR, AR, 1)
        for r in range(BK // AR):
            v = (x_sq + m3[r]) + e3[r]
            mask = v < run_min
            run_idx = jnp.where(mask, k3[r], run_idx)
            run_min = jnp.minimum(v, run_min)
    rows = AR
    while rows > 1:
        h = rows // 2
        a_min, b_min = run_min[:h], run_min[h:rows]
        a_idx, b_idx = run_idx[:h], run_idx[h:rows]
        take_b = (b_min < a_min) | ((b_min == a_min) & (b_idx < a_idx))
        run_min = jnp.where(take_b, b_min, a_min)
        run_idx = jnp.where(take_b, b_idx, a_idx)
        rows = h
    idx_ref[...] = run_idx.astype(jnp.int32)[None]


_argmin_call = pl.pallas_call(
    _argmin_body,
    grid=(N // BN,),
    in_specs=[
        pl.BlockSpec((D, BN), lambda i: (0, i)),
        pl.BlockSpec((K, D), lambda i: (0, 0)),
    ],
    out_specs=pl.BlockSpec((1, 1, BN), lambda i: (i, 0, 0)),
    out_shape=jax.ShapeDtypeStruct((N // BN, 1, BN), jnp.int32),
    scratch_shapes=[pltpu.VMEM((K, 1), jnp.float32),
                    pltpu.VMEM((K, 1), jnp.float32)],
)


_NC, _NS = 2, 16
_NW = _NC * _NS
_BPW = N // _NW


@functools.cache
def _make_gather_call():
    @functools.partial(
        pl.kernel,
        mesh=plsc.VectorSubcoreMesh(core_axis_name="c", subcore_axis_name="s"),
        out_type=(
            jax.ShapeDtypeStruct((N, D), jnp.float32),
            jax.ShapeDtypeStruct((N * D,), jnp.float32),
        ),
        scratch_types=[
            pltpu.VMEM((_BPW,), jnp.int32),
            pltpu.VMEM((_BPW, D), jnp.float32),
            pltpu.VMEM((_BPW * D,), jnp.float32),
            pltpu.VMEM((_BPW * D,), jnp.float32),
            pltpu.SemaphoreType.DMA,
        ],
        compiler_params=pltpu.CompilerParams(use_tc_tiling_on_sc=False),
    )
    def _gather_call(table_hbm, idx_hbm, x_hbm, out_hbm, z_hbm,
                     idx_v, rows_v, x_v, z_v, sem):
        wid = lax.axis_index("s") * _NC + lax.axis_index("c")
        base = wid * _BPW
        pltpu.sync_copy(idx_hbm.at[pl.ds(base, _BPW)], idx_v)
        gather = pltpu.async_copy(table_hbm.at[idx_v], rows_v, sem)
        pltpu.sync_copy(x_hbm.at[pl.ds(base * D, _BPW * D)], x_v)
        gather.wait()
        for r in range(_BPW):
            for c in range(D // 16):
                xv = x_v[r * D + c * 16:r * D + (c + 1) * 16]
                qv = rows_v[r, c * 16:(c + 1) * 16]
                z_v[r * D + c * 16:r * D + (c + 1) * 16] = (xv + qv) - xv
        pltpu.sync_copy(rows_v, out_hbm.at[pl.ds(base, _BPW)])
        pltpu.sync_copy(z_v, z_hbm.at[pl.ds(base * D, _BPW * D)])

    return _gather_call


def kernel(x, embeddings):
    idx = _argmin_call(x.reshape(N, D).T, embeddings).reshape(N)
    quantized, z_hat = _make_gather_call()(embeddings, idx, x)
    return (x, quantized.reshape(-1), z_hat, idx)


# speedup vs baseline: 1.0141x; 1.0141x over previous
import functools

import jax
import jax.numpy as jnp
from jax import lax
from jax.experimental import pallas as pl
from jax.experimental.pallas import tpu as pltpu
from jax.experimental.pallas import tpu_sc as plsc

N = 4096
K = 8192
D = 32

BN = 2048
BK = 2048


AR = 8


def _argmin_body(xt_ref, e_ref, idx_ref, esq_ref, kio_ref):
    @pl.when(pl.program_id(0) == 0)
    def _():
        e_all = e_ref[...]
        esq_ref[...] = jnp.sum(e_all * e_all, axis=1, keepdims=True)
        kio_ref[...] = lax.broadcasted_iota(
            jnp.int32, (K, 1), 0).astype(jnp.float32)

    xt = xt_ref[...]
    x2t = xt * -2.0
    x_sq = jnp.broadcast_to(
        jnp.sum(xt * xt, axis=0, keepdims=True), (AR, BN))
    run_min = jnp.full((AR, BN), jnp.inf, dtype=jnp.float32)
    run_idx = jnp.full((AR, BN), float(K), dtype=jnp.float32)
    for kc in range(K // BK):
        e_c = e_ref[kc * BK:(kc + 1) * BK, :]
        m2 = jnp.dot(e_c, x2t, preferred_element_type=jnp.float32)
        m3 = m2.reshape(BK // AR, AR, BN)
        e3 = esq_ref[kc * BK:(kc + 1) * BK, :].reshape(BK // AR, AR, 1)
        k3 = kio_ref[kc * BK:(kc + 1) * BK, :].reshape(BK // AR, AR, 1)
        for r in range(BK // AR):
            v = (x_sq + m3[r]) + e3[r]
            mask = v < run_min
            run_idx = jnp.where(mask, k3[r], run_idx)
            run_min = jnp.minimum(v, run_min)
    rows = AR
    while rows > 1:
        h = rows // 2
        a_min, b_min = run_min[:h], run_min[h:rows]
        a_idx, b_idx = run_idx[:h], run_idx[h:rows]
        take_b = (b_min < a_min) | ((b_min == a_min) & (b_idx < a_idx))
        run_min = jnp.where(take_b, b_min, a_min)
        run_idx = jnp.where(take_b, b_idx, a_idx)
        rows = h
    idx_ref[...] = run_idx.astype(jnp.int32)[None]


_argmin_call = pl.pallas_call(
    _argmin_body,
    grid=(N // BN,),
    in_specs=[
        pl.BlockSpec((D, BN), lambda i: (0, i)),
        pl.BlockSpec((K, D), lambda i: (0, 0)),
    ],
    out_specs=pl.BlockSpec((1, 1, BN), lambda i: (i, 0, 0)),
    out_shape=jax.ShapeDtypeStruct((N // BN, 1, BN), jnp.int32),
    scratch_shapes=[pltpu.VMEM((K, 1), jnp.float32),
                    pltpu.VMEM((K, 1), jnp.float32)],
)


_NC, _NS = 2, 16
_NW = _NC * _NS
_BPW = N // _NW


@functools.cache
def _make_gather_call():
    @functools.partial(
        pl.kernel,
        mesh=plsc.VectorSubcoreMesh(core_axis_name="c", subcore_axis_name="s"),
        out_type=(
            jax.ShapeDtypeStruct((N, D), jnp.float32),
            jax.ShapeDtypeStruct((N * D,), jnp.float32),
        ),
        scratch_types=[
            pltpu.VMEM((_BPW,), jnp.int32),
            pltpu.VMEM((_BPW, D), jnp.float32),
            pltpu.VMEM((_BPW * D,), jnp.float32),
            pltpu.VMEM((_BPW * D,), jnp.float32),
            pltpu.SemaphoreType.DMA,
        ],
        compiler_params=pltpu.CompilerParams(use_tc_tiling_on_sc=False),
    )
    def _gather_call(table_hbm, idx_hbm, x_hbm, out_hbm, z_hbm,
                     idx_v, rows_v, x_v, z_v, sem):
        wid = lax.axis_index("s") * _NC + lax.axis_index("c")
        base = wid * _BPW
        pltpu.sync_copy(idx_hbm.at[pl.ds(base, _BPW)], idx_v)
        gather = pltpu.async_copy(table_hbm.at[idx_v], rows_v, sem)
        pltpu.sync_copy(x_hbm.at[pl.ds(base * D, _BPW * D)], x_v)
        gather.wait()
        for r in range(_BPW):
            for c in range(D // 16):
                xv = x_v[r * D + c * 16:r * D + (c + 1) * 16]
                qv = rows_v[r, c * 16:(c + 1) * 16]
                z_v[r * D + c * 16:r * D + (c + 1) * 16] = (xv + qv) - xv
        pltpu.sync_copy(rows_v, out_hbm.at[pl.ds(base, _BPW)])
        pltpu.sync_copy(z_v, z_hbm.at[pl.ds(base * D, _BPW * D)])

    return _gather_call


def kernel(x, embeddings):
    idx = _argmin_call(x.reshape(N, D).T, embeddings).reshape(N)
    quantized, z_hat = _make_gather_call()(embeddings, idx, x)
    return (x, quantized.reshape(-1), z_hat, idx)
